# baseline (speedup 1.0000x reference)
import jax
import jax.numpy as jnp
from jax.experimental import pallas as pl

QB = 1024
LPAD = 1024


def _dist_argmin_kernel(x_ref, cm_ref, out_ref):
    xb = x_ref[...]
    feac = jnp.concatenate(
        [xb, jnp.ones((xb.shape[0], 1), dtype=xb.dtype)], axis=1)
    nrm = jnp.sqrt(jnp.sum(feac * feac, axis=1, keepdims=True))
    fea = feac / nrm
    d1 = feac.shape[1]
    dot = jnp.dot(fea, cm_ref[0:d1, :], preferred_element_type=jnp.float32)
    scores = cm_ref[d1:d1 + 1, :] - 2.0 * dot
    pred = jnp.argmin(scores, axis=1).astype(jnp.int32)
    out_ref[0, 0, :] = pred


def kernel(x, initc, labelset):
    q, d = x.shape
    l = labelset.shape[0]
    centers = jnp.take(initc, labelset, axis=0)
    cn = jnp.sum(centers * centers, axis=1)
    cmat = jnp.full((d + 2, LPAD), jnp.inf, dtype=jnp.float32)
    cmat = cmat.at[:d + 1, :].set(0.0)
    cmat = cmat.at[:d + 1, :l].set(centers.T)
    cmat = cmat.at[d + 1, :l].set(cn)

    nq = q // QB
    pred = pl.pallas_call(
        _dist_argmin_kernel,
        grid=(nq,),
        in_specs=[
            pl.BlockSpec((QB, d), lambda i: (i, 0)),
            pl.BlockSpec((d + 2, LPAD), lambda i: (0, 0)),
        ],
        out_specs=pl.BlockSpec((1, 1, QB), lambda i: (i, 0, 0)),
        out_shape=jax.ShapeDtypeStruct((nq, 1, QB), jnp.int32),
    )(x, cmat)
    pred = pred.reshape(q)
    return jnp.take(labelset, pred, axis=0)
